# baseline (speedup 1.0000x reference)
import dataclasses

import jax
import jax.numpy as jnp
from jax import lax
from jax.experimental import pallas as pl
from jax.experimental.pallas import tpu as pltpu
from jax.experimental.pallas import tpu_sc as plsc

T = 64
D_OUT = 128
NCORES = 2
NSUB = 16
NW = NCORES * NSUB
KCH = 4


def _stochastic_round_delays(log_delay, N, C):
    D = log_delay.shape[0]
    delay = jnp.concatenate([jnp.exp(log_delay), jnp.exp(log_delay[::-1])],
                            axis=1)
    db = jnp.broadcast_to(delay[None, None, :, :], (N, C, D, 2))
    fl = jnp.floor(db)
    p = db - fl
    bern = jax.random.bernoulli(jax.random.key(42), p)
    return jnp.where(bern, fl + 1.0, fl).astype(jnp.int32)


def _sc_body(x_hbm, rd_hbm, lw_hbm, out_hbm,
             xbuf, rdbuf, wbuf, tile0, tile1, sem0, sem1):
    ch_per_w = xbuf.shape[0]
    wid = lax.axis_index("c") * NSUB + lax.axis_index("s")
    base_ch = wid * ch_per_w

    pltpu.sync_copy(x_hbm.at[pl.ds(base_ch, ch_per_w)], xbuf)
    pltpu.sync_copy(rd_hbm.at[pl.ds(base_ch, ch_per_w)], rdbuf)
    pltpu.sync_copy(lw_hbm, wbuf)
    wv = jnp.exp(wbuf[...])

    @pl.loop(0, ch_per_w)
    def _(i):
        for j in range(2):
            for k in range(T // 16):
                sl = pl.ds(16 * k, 16)
                xbuf[i, j, sl] = xbuf[i, j, sl] * wv

    iota16 = lax.broadcasted_iota(jnp.int32, (16,), 0)
    tiles = (tile0, tile1)
    sems = (sem0, sem1)
    ngroups = ch_per_w // KCH

    def compute_channel(tile, i, cc):
        caps = []
        for j in range(2):
            m = jnp.max(xbuf[i, j, pl.ds(0, 16)])
            for k in range(1, T // 16):
                m = jnp.maximum(m, jnp.max(xbuf[i, j, pl.ds(16 * k, 16)]))
            best = jnp.int32(T)
            for k in range(T // 16):
                ck = xbuf[i, j, pl.ds(16 * k, 16)]
                idxs = jnp.where(ck == m, iota16 + 16 * k, jnp.int32(127))
                best = jnp.minimum(best, jnp.min(idxs))
            caps.append(jnp.int32(T - 1) - best)
        i16 = jnp.broadcast_to(i, (16,)).astype(jnp.int32)
        z16 = jnp.zeros((16,), jnp.int32)
        o16 = jnp.ones((16,), jnp.int32)
        for k8 in range(D_OUT // 16):
            dsl = pl.ds(16 * k8, 16)
            rd0 = jnp.minimum(rdbuf[i, 0, dsl], caps[0]) & (T - 1)
            rd1 = jnp.minimum(rdbuf[i, 1, dsl], caps[1]) & (T - 1)
            b0 = T - rd0
            b1 = T - rd1

            @pl.loop(0, T, step=4)
            def _(tt):
                for dt in range(4):
                    t = tt + dt
                    g0 = plsc.load_gather(xbuf, [i16, z16, (b0 + t) & (T - 1)])
                    g1 = plsc.load_gather(xbuf, [i16, o16, (b1 + t) & (T - 1)])
                    tile[t, cc, dsl] = g0 + g1

    @pl.loop(0, ngroups // 2)
    def _(gp):
        for b in range(2):
            g = gp * 2 + b
            ch0 = base_ch + g * KCH

            @pl.when(gp > 0)
            def _():
                pltpu.make_async_copy(
                    tiles[b], out_hbm.at[:, pl.ds(ch0, KCH), :], sems[b]).wait()

            @pl.loop(0, KCH)
            def _(cc):
                compute_channel(tiles[b], g * KCH + cc, cc)

            pltpu.async_copy(tiles[b], out_hbm.at[:, pl.ds(ch0, KCH), :],
                             sems[b])

    for b in range(2):
        pltpu.make_async_copy(
            tiles[b], out_hbm.at[:, pl.ds(base_ch, KCH), :], sems[b]).wait()


def kernel(input, log_delay, log_weight):
    Tt, N, C, _ = input.shape
    D = log_delay.shape[0]
    NC = N * C
    ch_per_w = NC // NW

    rd_pre = _stochastic_round_delays(log_delay, N, C)
    rdf = jnp.transpose(rd_pre, (0, 1, 3, 2)).reshape(NC, 2, D)
    xf = jnp.transpose(input, (1, 2, 3, 0)).reshape(NC, 2, Tt)
    lwv = jnp.full((16,), log_weight, jnp.float32)

    mesh = plsc.VectorSubcoreMesh(core_axis_name="c", subcore_axis_name="s")
    cp = pltpu.CompilerParams()
    if "needs_layout_passes" in pltpu.CompilerParams.__dataclass_fields__:
        cp = dataclasses.replace(cp, needs_layout_passes=False)
    run = pl.kernel(
        _sc_body,
        out_type=jax.ShapeDtypeStruct((Tt, NC, D), jnp.float32),
        mesh=mesh,
        scratch_types=[
            pltpu.VMEM((ch_per_w, 2, Tt), jnp.float32),
            pltpu.VMEM((ch_per_w, 2, D), jnp.int32),
            pltpu.VMEM((16,), jnp.float32),
            pltpu.VMEM((Tt, KCH, D), jnp.float32),
            pltpu.VMEM((Tt, KCH, D), jnp.float32),
            pltpu.SemaphoreType.DMA,
            pltpu.SemaphoreType.DMA,
        ],
        compiler_params=cp,
    )
    out = run(xf, rdf, lwv)
    return out.reshape(Tt, N, C, D)


# speedup vs baseline: 3.5597x; 3.5597x over previous
import jax
import jax.numpy as jnp
from jax.experimental import pallas as pl
from jax.experimental.pallas import tpu as pltpu


def _tc_body(xt_ref, rd_ref, lw_ref, mr_ref, out_ref, o1_ref):
    T = xt_ref.shape[3]
    C = xt_ref.shape[2]
    D = rd_ref.shape[3]

    w = jnp.exp(lw_ref[0, 0])
    mr = mr_ref[0, 0]

    def one_component(j, tgt):
        x = xt_ref[j, 0, :, :]
        m = jnp.max(x, axis=1, keepdims=True)
        tio = jax.lax.broadcasted_iota(jnp.int32, (C, T), 1)
        argm = jnp.min(jnp.where(x == m, tio, T), axis=1)
        cap = (T - 1) - argm
        rd = jnp.minimum(rd_ref[0, j, :, :], cap[:, None]) & (T - 1)
        xt = x.T
        tgt[...] = jnp.broadcast_to(xt[:, :, None], (T, C, D))
        bit = 1
        while bit < T:
            b = bit

            @pl.when(mr >= b)
            def _():
                o = tgt[...]
                rolled = jnp.concatenate([o[T - b:], o[:T - b]], axis=0)
                mask = ((rd & b) != 0)[None, :, :]
                tgt[...] = jnp.where(mask, rolled, o)

            bit *= 2

    out3 = out_ref.at[:, 0]
    one_component(0, out3)
    one_component(1, o1_ref)
    out3[...] = (out3[...] + o1_ref[...]) * w


def _stochastic_round_delays(log_delay, N, C):
    D = log_delay.shape[0]
    delay = jnp.concatenate([jnp.exp(log_delay), jnp.exp(log_delay[::-1])],
                            axis=1)
    db = jnp.broadcast_to(delay[None, None, :, :], (N, C, D, 2))
    fl = jnp.floor(db)
    p = db - fl
    bern = jax.random.bernoulli(jax.random.key(42), p)
    return jnp.where(bern, fl + 1.0, fl).astype(jnp.int32)


def kernel(input, log_delay, log_weight):
    T, N, C, _ = input.shape
    D = log_delay.shape[0]

    rd_pre = _stochastic_round_delays(log_delay, N, C)
    rd_t = jnp.transpose(rd_pre, (0, 3, 1, 2))
    xt = jnp.transpose(input, (3, 1, 2, 0))
    lw = jnp.reshape(log_weight, (1, 1)).astype(jnp.float32)
    mr = jnp.minimum(jnp.max(rd_t), T - 1).reshape(1, 1).astype(jnp.int32)

    out = pl.pallas_call(
        _tc_body,
        grid=(N,),
        in_specs=[
            pl.BlockSpec((2, 1, C, T), lambda n: (0, n, 0, 0)),
            pl.BlockSpec((1, 2, C, D), lambda n: (n, 0, 0, 0)),
            pl.BlockSpec(memory_space=pltpu.SMEM),
            pl.BlockSpec(memory_space=pltpu.SMEM),
        ],
        out_specs=pl.BlockSpec((T, 1, C, D), lambda n: (0, n, 0, 0)),
        out_shape=jax.ShapeDtypeStruct((T, N, C, D), jnp.float32),
        scratch_shapes=[pltpu.VMEM((T, C, D), jnp.float32)],
    )(xt, rd_t, lw, mr)
    return out
